# baseline (speedup 1.0000x reference)
import jax
import jax.numpy as jnp
from jax.experimental import pallas as pl
from jax.experimental.pallas import tpu as pltpu

_RING = 16
_SPLIT = 2


def _copy_body(starts_ref, curr_ref, hist_ref, out_ref, buf, in_sem, out_sem):
    B, H = hist_ref.shape[0], hist_ref.shape[1]
    C = hist_ref.shape[2]
    CS = C // _SPLIT
    n = B * H * _SPLIT

    def src_at(i):
        r, c = divmod(i, _SPLIT)
        bs, k = divmod(r, H)
        if k == 0:
            return curr_ref.at[pl.ds(starts_ref[bs], 1), pl.ds(c * CS, CS)]
        return hist_ref.at[bs, pl.ds(k, 1), pl.ds(c * CS, CS)]

    def dst_at(i):
        r, c = divmod(i, _SPLIT)
        return out_ref.at[pl.ds(r, 1), pl.ds(c * CS, CS)]

    def start_in(i):
        pltpu.make_async_copy(src_at(i), buf.at[pl.ds(i % _RING, 1)], in_sem.at[i]).start()

    def wait_in(i):
        pltpu.make_async_copy(src_at(i), buf.at[pl.ds(i % _RING, 1)], in_sem.at[i]).wait()

    def start_out(i):
        pltpu.make_async_copy(
            buf.at[pl.ds(i % _RING, 1)], dst_at(i), out_sem.at[i]
        ).start()

    def wait_out(i):
        pltpu.make_async_copy(
            buf.at[pl.ds(i % _RING, 1)], dst_at(i), out_sem.at[i]
        ).wait()

    for i in range(_RING):
        start_in(i)
    for i in range(n):
        wait_in(i)
        start_out(i)
        j = i + _RING
        if j < n:
            wait_out(j - _RING)
            start_in(j)
    for i in range(n - _RING, n):
        wait_out(i)


def kernel(feat_curr, feat_history, record_len):
    B, H, C, Hh, W = feat_history.shape
    starts = (jnp.cumsum(record_len) - record_len).astype(jnp.int32)

    feat_final = pl.pallas_call(
        _copy_body,
        in_specs=[
            pl.BlockSpec(memory_space=pltpu.SMEM),
            pl.BlockSpec(memory_space=pltpu.MemorySpace.HBM),
            pl.BlockSpec(memory_space=pltpu.MemorySpace.HBM),
        ],
        out_specs=pl.BlockSpec(memory_space=pltpu.MemorySpace.HBM),
        out_shape=jax.ShapeDtypeStruct((B * H, C, Hh, W), feat_curr.dtype),
        scratch_shapes=[
            pltpu.VMEM((_RING, C // _SPLIT, Hh, W), feat_curr.dtype),
            pltpu.SemaphoreType.DMA((B * H * _SPLIT,)),
            pltpu.SemaphoreType.DMA((B * H * _SPLIT,)),
        ],
    )(starts, feat_curr, feat_history)

    offset_loss = jnp.zeros((1,), dtype=feat_final.dtype)
    return (feat_final, offset_loss)
